# speedup vs baseline: 2.9527x; 2.9527x over previous
import functools
import math

import jax
import jax.numpy as jnp
from jax import lax
from jax.experimental import pallas as pl
from jax.experimental.pallas import tpu as pltpu
from jax.experimental.pallas import tpu_sc as plsc

VOCAB = 100000
D_MODEL = 128
BATCH = 4096
HIST = 50

NC = 2
NS = 16
NW = NC * NS
B_TOTAL = BATCH * HIST
B_PER_W = B_TOTAL // NW
CHUNK = 128
NCHUNK = B_PER_W // CHUNK
NBUF = 5
SCALE = math.sqrt(D_MODEL)

_mesh = plsc.VectorSubcoreMesh(core_axis_name="c", subcore_axis_name="s")


@functools.partial(
    pl.kernel,
    mesh=_mesh,
    out_type=jax.ShapeDtypeStruct((NW, NCHUNK, CHUNK, D_MODEL), jnp.float32),
    scratch_types=[
        pltpu.VMEM((NCHUNK, CHUNK), jnp.int32),
        pltpu.VMEM((NBUF, CHUNK, D_MODEL), jnp.float32),
        pltpu.SemaphoreType.DMA,
        pltpu.SemaphoreType.DMA,
    ],
)
def _emb_sc(x_hbm, w_hbm, out_hbm, idx_v, rows_v, gsem, osem):
    wid = lax.axis_index("s") * NC + lax.axis_index("c")

    pltpu.sync_copy(x_hbm.at[wid], idx_v)

    def gather_start(c, b):
        pltpu.async_copy(w_hbm.at[idx_v.at[c]], rows_v.at[b], gsem)

    def gather_wait(c, b):
        pltpu.make_async_copy(w_hbm.at[idx_v.at[c]], rows_v.at[b], gsem).wait()

    def out_start(c, b):
        pltpu.async_copy(rows_v.at[b], out_hbm.at[wid, c], osem)

    def out_wait(c, b):
        pltpu.make_async_copy(rows_v.at[b], out_hbm.at[wid, c], osem).wait()

    def scale_buf(b):
        rows = rows_v.at[b]

        def body(j, _):
            for i in range(D_MODEL // 16):
                sl = pl.ds(16 * i, 16)
                rows[j, sl] = rows[j, sl] * SCALE
            return 0

        lax.fori_loop(0, CHUNK, body, 0, unroll=4)

    for b in range(NBUF):
        gather_start(b, b)

    def outer(c0, _):
        for b in range(NBUF):
            c = c0 * NBUF + b
            gather_wait(c, b)
            scale_buf(b)
            out_start(c, b)
            nxt = c + NBUF

            @pl.when(nxt < NCHUNK)
            def _():
                out_wait(c, b)
                gather_start(nxt, b)

        return 0

    lax.fori_loop(0, NCHUNK // NBUF, outer, 0)

    for b in range(NBUF):
        out_wait(NCHUNK - NBUF + b, b)


def kernel(x, weight):
    xf = x.reshape(NW, NCHUNK, CHUNK)
    out = _emb_sc(xf, weight)
    return out.reshape(BATCH, HIST, D_MODEL)
